# baseline (speedup 1.0000x reference)
import functools

import jax
import jax.numpy as jnp
from jax import lax
from jax.experimental import pallas as pl
from jax.experimental.pallas import tpu as pltpu
from jax.experimental.pallas import tpu_sc as plsc

N_ROWS = 1_000_000
D = 64
B = 4096 * 26
LANES = 16
DV = D // LANES

_info = plsc.get_sparse_core_info()
_NC, _NS = _info.num_cores, _info.num_subcores
NW = _NC * _NS
BPW = B // NW
NCHUNK = 4
C = BPW // NCHUNK


def _rsqrt_newton(s2v):
    ib = lax.bitcast_convert_type(s2v, jnp.int32)
    ib = jnp.int32(0x5F3759DF) - lax.shift_right_logical(ib, 1)
    y = lax.bitcast_convert_type(ib, jnp.float32)
    for _ in range(3):
        y = y * (1.5 - 0.5 * s2v * y * y)
    return y


_mesh = plsc.VectorSubcoreMesh(core_axis_name="c", subcore_axis_name="s")


@functools.partial(
    pl.kernel,
    out_type=jax.ShapeDtypeStruct((B, D), jnp.float32),
    mesh=_mesh,
    compiler_params=pltpu.CompilerParams(use_tc_tiling_on_sc=False),
    scratch_types=[
        pltpu.VMEM((BPW,), jnp.int32),
        pltpu.VMEM((C, D), jnp.float32),
        pltpu.VMEM((C, D), jnp.float32),
        pltpu.SemaphoreType.DMA,
        pltpu.SemaphoreType.DMA,
        pltpu.SemaphoreType.DMA,
        pltpu.SemaphoreType.DMA,
    ],
)
def _embed_gather(idx_hbm, table_hbm, out_hbm, idx_v, buf0, buf1,
                  gsem0, gsem1, osem0, osem1):
    wid = lax.axis_index("s") * _NC + lax.axis_index("c")
    base = wid * BPW

    lanes = lax.iota(jnp.int32, LANES)
    perms = [(lanes + (1 << k)) & (LANES - 1) for k in range(4)]
    dnums = lax.GatherDimensionNumbers(
        offset_dims=(), collapsed_slice_dims=(0,), start_index_map=(0,))

    def _shuffle(v, p):
        return lax.gather(v, p.reshape(LANES, 1), dnums, slice_sizes=(1,),
                          mode=lax.GatherScatterMode.PROMISE_IN_BOUNDS)
    bufs = (buf0, buf1)
    gsems = (gsem0, gsem1)
    osems = (osem0, osem1)

    pltpu.sync_copy(idx_hbm.at[pl.ds(base, BPW)], idx_v)

    gcopy = [None, None]
    ocopy = [None, None]
    gcopy[0] = pltpu.async_copy(
        table_hbm.at[idx_v.at[pl.ds(0, C)]], bufs[0], gsems[0])

    for g in range(NCHUNK):
        b = g % 2
        buf = bufs[b]
        gcopy[b].wait()

        if g + 1 < NCHUNK:
            nb = (g + 1) % 2
            if ocopy[nb] is not None:
                ocopy[nb].wait()
            gcopy[nb] = pltpu.async_copy(
                table_hbm.at[idx_v.at[pl.ds((g + 1) * C, C)]],
                bufs[nb], gsems[nb])

        def mx_body(i, m):
            for c in range(DV):
                v = buf[i, pl.ds(c * LANES, LANES)]
                m = jnp.maximum(m, v * v)
            return m

        m = lax.fori_loop(0, C, mx_body, jnp.zeros((LANES,), jnp.float32))
        for p in perms:
            m = jnp.maximum(m, _shuffle(m, p))
        mmax = lax.squeeze(lax.slice(m, (0,), (1,)), (0,))

        @pl.when(mmax * jnp.float32(D) > 1.0)
        def _fixup():
            def row_body(i, carry):
                acc = jnp.zeros((LANES,), jnp.float32)
                for c in range(DV):
                    v = buf[i, pl.ds(c * LANES, LANES)]
                    acc = acc + v * v
                s2v = acc
                for p in perms:
                    s2v = s2v + _shuffle(s2v, p)
                y = _rsqrt_newton(s2v)
                scale = jnp.where(s2v > 1.0, y, jnp.float32(1.0))
                for c in range(DV):
                    buf[i, pl.ds(c * LANES, LANES)] = (
                        buf[i, pl.ds(c * LANES, LANES)] * scale)
                return carry

            lax.fori_loop(0, C, row_body, 0)

        ocopy[b] = pltpu.async_copy(
            buf, out_hbm.at[pl.ds(base + g * C, C)], osems[b])

    ocopy[0].wait()
    ocopy[1].wait()


def kernel(x, table):
    xf = x.reshape(-1).astype(jnp.int32)
    out = _embed_gather(xf, table)
    return out.reshape(x.shape + (table.shape[1],))


# speedup vs baseline: 1.3570x; 1.3570x over previous
import functools

import jax
import jax.numpy as jnp
from jax import lax
from jax.experimental import pallas as pl
from jax.experimental.pallas import tpu as pltpu
from jax.experimental.pallas import tpu_sc as plsc

N_ROWS = 1_000_000
D = 64
B = 4096 * 26
LANES = 16
DV = D // LANES

_info = plsc.get_sparse_core_info()
_NC, _NS = _info.num_cores, _info.num_subcores
NW = _NC * _NS
BPW = B // NW
NCHUNK = 13
C = BPW // NCHUNK
G = 16
NG = C // G
DRAIN = 2


def _rsqrt_newton(s2v):
    ib = lax.bitcast_convert_type(s2v, jnp.int32)
    ib = jnp.int32(0x5F3759DF) - lax.shift_right_logical(ib, 1)
    y = lax.bitcast_convert_type(ib, jnp.float32)
    for _ in range(3):
        y = y * (1.5 - 0.5 * s2v * y * y)
    return y


_mesh = plsc.VectorSubcoreMesh(core_axis_name="c", subcore_axis_name="s")


@functools.partial(
    pl.kernel,
    out_type=jax.ShapeDtypeStruct((B, D), jnp.float32),
    mesh=_mesh,
    compiler_params=pltpu.CompilerParams(use_tc_tiling_on_sc=True),
    scratch_types=[
        pltpu.VMEM((BPW,), jnp.int32),
        pltpu.VMEM((C, D), jnp.float32),
        pltpu.VMEM((C, D), jnp.float32),
        pltpu.SemaphoreType.DMA,
        pltpu.SemaphoreType.DMA,
        pltpu.SemaphoreType.DMA,
        pltpu.SemaphoreType.DMA,
    ],
)
def _embed_gather(idx_hbm, table_hbm, out_hbm, idx_v, gbuf0, gbuf1,
                  gsem0, gsem1, osem0, osem1):
    wid = lax.axis_index("s") * _NC + lax.axis_index("c")
    base = wid * BPW
    gbufs = (gbuf0, gbuf1)
    gsems = (gsem0, gsem1)
    osems = (osem0, osem1)

    lanes = lax.iota(jnp.int32, LANES)
    perms = [(lanes + (1 << k)) & (LANES - 1) for k in range(4)]
    dnums = lax.GatherDimensionNumbers(
        offset_dims=(), collapsed_slice_dims=(0,), start_index_map=(0,))

    def _shuffle(v, p):
        return lax.gather(v, p.reshape(LANES, 1), dnums, slice_sizes=(1,),
                          mode=lax.GatherScatterMode.PROMISE_IN_BOUNDS)

    pltpu.sync_copy(idx_hbm.at[pl.ds(base, BPW)], idx_v)

    ocopy = [None, None]

    def gather_chunk(g, gbuf, gsem):
        def issue_group(k):
            idx16 = idx_v[pl.ds(g * C + k * G, G)]
            for j in range(G):
                iq = lax.squeeze(lax.slice(idx16, (j,), (j + 1,)), (0,))
                pltpu.async_copy(table_hbm.at[pl.ds(iq, 1)],
                                 gbuf.at[pl.ds(k * G + j, 1)], gsem)

        def drain_group():
            pltpu.make_async_copy(table_hbm.at[pl.ds(0, G)],
                                  gbuf.at[pl.ds(0, G)], gsem).wait()

        def grp_body(k, carry):
            issue_group(k)

            @pl.when(k >= DRAIN)
            def _():
                drain_group()

            return carry

        lax.fori_loop(0, NG, grp_body, 0)
        for _ in range(DRAIN):
            drain_group()

    for g in range(NCHUNK):
        b = g % 2
        gbuf = gbufs[b]

        if ocopy[b] is not None:
            ocopy[b].wait()

        gather_chunk(g, gbuf, gsems[b])

        def mx_body(i, m):
            for c in range(DV):
                v = gbuf[i, pl.ds(c * LANES, LANES)]
                m = jnp.maximum(m, v * v)
            return m

        m = lax.fori_loop(0, C, mx_body, jnp.zeros((LANES,), jnp.float32))
        for p in perms:
            m = jnp.maximum(m, _shuffle(m, p))
        mmax = lax.squeeze(lax.slice(m, (0,), (1,)), (0,))

        @pl.when(mmax * jnp.float32(D) > 1.0)
        def _fixup():
            def row_body(i, carry):
                acc = jnp.zeros((LANES,), jnp.float32)
                for c in range(DV):
                    v = gbuf[i, pl.ds(c * LANES, LANES)]
                    acc = acc + v * v
                s2v = acc
                for p in perms:
                    s2v = s2v + _shuffle(s2v, p)
                y = _rsqrt_newton(s2v)
                scale = jnp.where(s2v > 1.0, y, jnp.float32(1.0))
                for c in range(DV):
                    gbuf[i, pl.ds(c * LANES, LANES)] = (
                        gbuf[i, pl.ds(c * LANES, LANES)] * scale)
                return carry

            lax.fori_loop(0, C, row_body, 0)

        ocopy[b] = pltpu.async_copy(
            gbuf, out_hbm.at[pl.ds(base + g * C, C)], osems[b])

    ocopy[0].wait()
    ocopy[1].wait()


def kernel(x, table):
    xf = x.reshape(-1).astype(jnp.int32)
    out = _embed_gather(xf, table)
    return out.reshape(x.shape + (table.shape[1],))


# speedup vs baseline: 2.0504x; 1.5110x over previous
import functools

import jax
import jax.numpy as jnp
from jax import lax
from jax.experimental import pallas as pl
from jax.experimental.pallas import tpu as pltpu
from jax.experimental.pallas import tpu_sc as plsc

N_ROWS = 1_000_000
D = 64
NB = 4096
NJ = 26
B = NB * NJ
LANES = 16
DV = D // LANES

_info = plsc.get_sparse_core_info()
_NC, _NS = _info.num_cores, _info.num_subcores
NW = _NC * _NS
BPT = NB // NW
BPW = B // NW
CB = 8
NCHUNK = BPT // CB
DRAIN_B = 4


def _rsqrt_newton(s2v):
    ib = lax.bitcast_convert_type(s2v, jnp.int32)
    ib = jnp.int32(0x5F3759DF) - lax.shift_right_logical(ib, 1)
    y = lax.bitcast_convert_type(ib, jnp.float32)
    for _ in range(3):
        y = y * (1.5 - 0.5 * s2v * y * y)
    return y


_mesh = plsc.VectorSubcoreMesh(core_axis_name="c", subcore_axis_name="s")


@functools.partial(
    pl.kernel,
    out_type=jax.ShapeDtypeStruct((NB, NJ, D), jnp.float32),
    mesh=_mesh,
    compiler_params=pltpu.CompilerParams(use_tc_tiling_on_sc=True),
    scratch_types=[
        pltpu.VMEM((BPW + LANES,), jnp.int32),
        pltpu.VMEM((CB, NJ, D), jnp.float32),
        pltpu.VMEM((CB, NJ, D), jnp.float32),
        pltpu.SemaphoreType.DMA,
        pltpu.SemaphoreType.DMA,
        pltpu.SemaphoreType.DMA,
        pltpu.SemaphoreType.DMA,
    ],
)
def _embed_gather(idx_hbm, table_hbm, out_hbm, idx_v, gbuf0, gbuf1,
                  gsem0, gsem1, osem0, osem1):
    wid = lax.axis_index("s") * _NC + lax.axis_index("c")
    base = wid * BPW
    b_lo = wid * BPT
    gbufs = (gbuf0, gbuf1)
    gsems = (gsem0, gsem1)
    osems = (osem0, osem1)

    lanes = lax.iota(jnp.int32, LANES)
    perms = [(lanes + (1 << k)) & (LANES - 1) for k in range(4)]
    dnums = lax.GatherDimensionNumbers(
        offset_dims=(), collapsed_slice_dims=(0,), start_index_map=(0,))

    def _shuffle(v, p):
        return lax.gather(v, p.reshape(LANES, 1), dnums, slice_sizes=(1,),
                          mode=lax.GatherScatterMode.PROMISE_IN_BOUNDS)

    pltpu.sync_copy(idx_hbm.at[pl.ds(base, BPW)], idx_v.at[pl.ds(0, BPW)])

    def chunk_body(ci, gbuf, gsem, osem):
        b0 = ci * CB

        @pl.when(ci >= 2)
        def _():
            pltpu.make_async_copy(out_hbm.at[pl.ds(0, CB)], gbuf, osem).wait()

        def drain_b():
            for _ in range(NJ):
                pltpu.make_async_copy(table_hbm.at[0, pl.ds(0, 1)],
                                      gbuf.at[0, pl.ds(0, 1)], gsem).wait()

        def issue_b(bb, carry):
            off = (b0 + bb) * NJ
            v0 = idx_v[pl.ds(off, LANES)]
            v1 = idx_v[pl.ds(off + LANES, LANES)]
            for jj in range(NJ):
                src = v0 if jj < LANES else v1
                iq = lax.squeeze(
                    lax.slice(src, (jj % LANES,), (jj % LANES + 1,)), (0,))
                q = lax.shift_right_logical(iq, 3)
                s = iq & 7
                pltpu.async_copy(table_hbm.at[q, pl.ds(s, 1)],
                                 gbuf.at[bb, pl.ds(jj, 1)], gsem)

            @pl.when(bb >= DRAIN_B)
            def _():
                drain_b()

            return carry

        lax.fori_loop(0, CB, issue_b, 0)
        for _ in range(DRAIN_B):
            drain_b()

        m = jnp.zeros((LANES,), jnp.float32)
        for bb in range(CB):
            def mx_body(jj, mm, bb=bb):
                for c in range(DV):
                    v = gbuf[bb, jj, pl.ds(c * LANES, LANES)]
                    mm = jnp.maximum(mm, v * v)
                return mm

            m = lax.fori_loop(0, NJ, mx_body, m)
        for p in perms:
            m = jnp.maximum(m, _shuffle(m, p))
        mmax = lax.squeeze(lax.slice(m, (0,), (1,)), (0,))

        @pl.when(mmax * jnp.float32(D) > 1.0)
        def _fixup():
            for bb in range(CB):
                def row_body(jj, carry, bb=bb):
                    acc = jnp.zeros((LANES,), jnp.float32)
                    for c in range(DV):
                        v = gbuf[bb, jj, pl.ds(c * LANES, LANES)]
                        acc = acc + v * v
                    s2v = acc
                    for p in perms:
                        s2v = s2v + _shuffle(s2v, p)
                    y = _rsqrt_newton(s2v)
                    scale = jnp.where(s2v > 1.0, y, jnp.float32(1.0))
                    for c in range(DV):
                        gbuf[bb, jj, pl.ds(c * LANES, LANES)] = (
                            gbuf[bb, jj, pl.ds(c * LANES, LANES)] * scale)
                    return carry

                lax.fori_loop(0, NJ, row_body, 0)

        return pltpu.async_copy(
            gbuf, out_hbm.at[pl.ds(b_lo + b0, CB)], osem)

    def pair_body(g, carry):
        ci = 2 * g
        chunk_body(ci, gbufs[0], gsems[0], osems[0])
        chunk_body(ci + 1, gbufs[1], gsems[1], osems[1])
        return carry

    lax.fori_loop(0, NCHUNK // 2, pair_body, 0)

    pltpu.make_async_copy(out_hbm.at[pl.ds(0, CB)], gbufs[0], osems[0]).wait()
    pltpu.make_async_copy(out_hbm.at[pl.ds(0, CB)], gbufs[1], osems[1]).wait()


def kernel(x, table):
    xf = x.reshape(-1).astype(jnp.int32)
    out = _embed_gather(xf, table.reshape(N_ROWS // 8, 8, D))
    return out.reshape(x.shape + (table.shape[1],))


# speedup vs baseline: 2.0593x; 1.0043x over previous
import functools

import jax
import jax.numpy as jnp
from jax import lax
from jax.experimental import pallas as pl
from jax.experimental.pallas import tpu as pltpu
from jax.experimental.pallas import tpu_sc as plsc

N_ROWS = 1_000_000
D = 64
NB = 4096
NJ = 26
B = NB * NJ
LANES = 16
DV = D // LANES

_info = plsc.get_sparse_core_info()
_NC, _NS = _info.num_cores, _info.num_subcores
NW = _NC * _NS
BPT = NB // NW
BPW = B // NW
CB = 8
NCHUNK = BPT // CB
DRAIN_B = 6


def _rsqrt_newton(s2v):
    ib = lax.bitcast_convert_type(s2v, jnp.int32)
    ib = jnp.int32(0x5F3759DF) - lax.shift_right_logical(ib, 1)
    y = lax.bitcast_convert_type(ib, jnp.float32)
    for _ in range(3):
        y = y * (1.5 - 0.5 * s2v * y * y)
    return y


_mesh = plsc.VectorSubcoreMesh(core_axis_name="c", subcore_axis_name="s")


@functools.partial(
    pl.kernel,
    out_type=jax.ShapeDtypeStruct((NB, NJ, D), jnp.float32),
    mesh=_mesh,
    compiler_params=pltpu.CompilerParams(use_tc_tiling_on_sc=True),
    scratch_types=[
        pltpu.VMEM((BPW + LANES,), jnp.int32),
        pltpu.VMEM((CB, NJ, D), jnp.float32),
        pltpu.VMEM((CB, NJ, D), jnp.float32),
        pltpu.SemaphoreType.DMA,
        pltpu.SemaphoreType.DMA,
        pltpu.SemaphoreType.DMA,
        pltpu.SemaphoreType.DMA,
    ],
)
def _embed_gather(idx_hbm, table_hbm, out_hbm, idx_v, gbuf0, gbuf1,
                  gsem0, gsem1, osem0, osem1):
    wid = lax.axis_index("s") * _NC + lax.axis_index("c")
    base = wid * BPW
    b_lo = wid * BPT
    gbufs = (gbuf0, gbuf1)
    gsems = (gsem0, gsem1)
    osems = (osem0, osem1)

    lanes = lax.iota(jnp.int32, LANES)
    perms = [(lanes + (1 << k)) & (LANES - 1) for k in range(4)]
    dnums = lax.GatherDimensionNumbers(
        offset_dims=(), collapsed_slice_dims=(0,), start_index_map=(0,))

    def _shuffle(v, p):
        return lax.gather(v, p.reshape(LANES, 1), dnums, slice_sizes=(1,),
                          mode=lax.GatherScatterMode.PROMISE_IN_BOUNDS)

    pltpu.sync_copy(idx_hbm.at[pl.ds(base, BPW)], idx_v.at[pl.ds(0, BPW)])

    def chunk_body(ci, gbuf, gsem, osem):
        b0 = ci * CB

        @pl.when(ci >= 2)
        def _():
            pltpu.make_async_copy(out_hbm.at[pl.ds(0, CB)], gbuf, osem).wait()

        def drain_b():
            for _ in range(NJ):
                pltpu.make_async_copy(table_hbm.at[0, pl.ds(0, 1)],
                                      gbuf.at[0, pl.ds(0, 1)], gsem).wait()

        def issue_b(bb, carry):
            off = (b0 + bb) * NJ
            v0 = idx_v[pl.ds(off, LANES)]
            v1 = idx_v[pl.ds(off + LANES, LANES)]
            for jj in range(NJ):
                src = v0 if jj < LANES else v1
                iq = lax.squeeze(
                    lax.slice(src, (jj % LANES,), (jj % LANES + 1,)), (0,))
                q = lax.shift_right_logical(iq, 3)
                s = iq & 7
                pltpu.async_copy(table_hbm.at[q, pl.ds(s, 1)],
                                 gbuf.at[bb, pl.ds(jj, 1)], gsem)

            @pl.when(bb >= DRAIN_B)
            def _():
                drain_b()

            return carry

        lax.fori_loop(0, CB, issue_b, 0)
        for _ in range(DRAIN_B):
            drain_b()

        m = jnp.zeros((LANES,), jnp.float32)
        for bb in range(CB):
            def mx_body(jj, mm, bb=bb):
                for c in range(DV):
                    v = gbuf[bb, jj, pl.ds(c * LANES, LANES)]
                    mm = jnp.maximum(mm, v * v)
                return mm

            m = lax.fori_loop(0, NJ, mx_body, m)
        for p in perms:
            m = jnp.maximum(m, _shuffle(m, p))
        mmax = lax.squeeze(lax.slice(m, (0,), (1,)), (0,))

        @pl.when(mmax * jnp.float32(D) > 1.0)
        def _fixup():
            for bb in range(CB):
                def row_body(jj, carry, bb=bb):
                    acc = jnp.zeros((LANES,), jnp.float32)
                    for c in range(DV):
                        v = gbuf[bb, jj, pl.ds(c * LANES, LANES)]
                        acc = acc + v * v
                    s2v = acc
                    for p in perms:
                        s2v = s2v + _shuffle(s2v, p)
                    y = _rsqrt_newton(s2v)
                    scale = jnp.where(s2v > 1.0, y, jnp.float32(1.0))
                    for c in range(DV):
                        gbuf[bb, jj, pl.ds(c * LANES, LANES)] = (
                            gbuf[bb, jj, pl.ds(c * LANES, LANES)] * scale)
                    return carry

                lax.fori_loop(0, NJ, row_body, 0)

        return pltpu.async_copy(
            gbuf, out_hbm.at[pl.ds(b_lo + b0, CB)], osem)

    def pair_body(g, carry):
        ci = 2 * g
        chunk_body(ci, gbufs[0], gsems[0], osems[0])
        chunk_body(ci + 1, gbufs[1], gsems[1], osems[1])
        return carry

    lax.fori_loop(0, NCHUNK // 2, pair_body, 0)

    pltpu.make_async_copy(out_hbm.at[pl.ds(0, CB)], gbufs[0], osems[0]).wait()
    pltpu.make_async_copy(out_hbm.at[pl.ds(0, CB)], gbufs[1], osems[1]).wait()


def kernel(x, table):
    xf = x.reshape(-1).astype(jnp.int32)
    out = _embed_gather(xf, table.reshape(N_ROWS // 8, 8, D))
    return out.reshape(x.shape + (table.shape[1],))
